# baseline (speedup 1.0000x reference)
import functools

import jax
import jax.numpy as jnp
import numpy as np
from jax import lax
from jax.experimental import pallas as pl
from jax.experimental.pallas import tpu as pltpu
from jax.experimental.pallas import tpu_sc as plsc

NUM_SEGS = 512
ROWS_PER_SEG = 625
D = 128
NLANES = 16
NVEC = D // NLANES

NUM_CORES = 2
NUM_SUBCORES = 16
NUM_WORKERS = NUM_CORES * NUM_SUBCORES

TC_SEGS = 288
SC_SEGS = NUM_SEGS - TC_SEGS
SEGS_PER_WORKER = SC_SEGS // NUM_WORKERS

NCHUNK = 5
CHUNK_ROWS = ROWS_PER_SEG // NCHUNK
CHUNK_ELEMS = CHUNK_ROWS * D

TC_BLOCK_SEGS = 8


def _sc_segment_sum(edges_flat):
    mesh = plsc.VectorSubcoreMesh(core_axis_name="c", subcore_axis_name="s")

    @functools.partial(
        pl.kernel,
        out_type=jax.ShapeDtypeStruct((SC_SEGS * D,), jnp.float32),
        mesh=mesh,
        scratch_types=(
            [pltpu.VMEM((CHUNK_ELEMS,), jnp.float32) for _ in range(NCHUNK)]
            + [pltpu.VMEM((SEGS_PER_WORKER * D,), jnp.float32)]
            + [pltpu.SemaphoreType.DMA for _ in range(NCHUNK)]
        ),
    )
    def body(edges_hbm, out_hbm, *scratch):
        bufs = scratch[:NCHUNK]
        stage = scratch[NCHUNK]
        sems = scratch[NCHUNK + 1:]
        cid = lax.axis_index("c")
        sid = lax.axis_index("s")
        wid = sid * NUM_CORES + cid
        base_seg = TC_SEGS + wid * SEGS_PER_WORKER

        def chunk_copy(seg, b):
            e0 = (seg * ROWS_PER_SEG + b * CHUNK_ROWS) * D
            return pltpu.make_async_copy(
                edges_hbm.at[pl.ds(e0, CHUNK_ELEMS)], bufs[b], sems[b]
            )

        for b in range(NCHUNK):
            chunk_copy(base_seg, b).start()

        def seg_body(s, carry):
            seg = base_seg + s
            acc = tuple(jnp.zeros((NLANES,), jnp.float32) for _ in range(NVEC))
            for b in range(NCHUNK):
                chunk_copy(seg, b).wait()

                def row_body(r, a, b=b):
                    base = r * D
                    return tuple(
                        a[j] + bufs[b][pl.ds(base + j * NLANES, NLANES)]
                        for j in range(NVEC)
                    )

                acc = lax.fori_loop(0, CHUNK_ROWS, row_body, acc)

                @pl.when(s < SEGS_PER_WORKER - 1)
                def _(b=b):
                    chunk_copy(seg + 1, b).start()

            for j in range(NVEC):
                stage[pl.ds(s * D + j * NLANES, NLANES)] = acc[j]
            return carry

        lax.fori_loop(0, SEGS_PER_WORKER, seg_body, 0)
        local_out0 = (base_seg - TC_SEGS) * D
        pltpu.sync_copy(stage, out_hbm.at[pl.ds(local_out0, SEGS_PER_WORKER * D)])

    return body(edges_flat)


def _tc_body(x_ref, m_ref, o_ref):
    o_ref[...] = jnp.dot(
        m_ref[...], x_ref[...], preferred_element_type=jnp.float32
    )


_ONEHOT = np.kron(
    np.eye(TC_BLOCK_SEGS, dtype=np.float32), np.ones((1, ROWS_PER_SEG), np.float32)
)


def _tc_segment_sum(edges2d):
    return pl.pallas_call(
        _tc_body,
        grid=(TC_SEGS // TC_BLOCK_SEGS,),
        in_specs=[
            pl.BlockSpec((TC_BLOCK_SEGS * ROWS_PER_SEG, D), lambda i: (i, 0)),
            pl.BlockSpec(
                (TC_BLOCK_SEGS, TC_BLOCK_SEGS * ROWS_PER_SEG), lambda i: (0, 0)
            ),
        ],
        out_specs=pl.BlockSpec((TC_BLOCK_SEGS, D), lambda i: (i, 0)),
        out_shape=jax.ShapeDtypeStruct((TC_SEGS, D), jnp.float32),
    )(edges2d, jnp.asarray(_ONEHOT))


def kernel(edges, n_edge, num_hypergraphs):
    del n_edge, num_hypergraphs
    sc_out = _sc_segment_sum(edges.reshape(-1))
    tc_out = _tc_segment_sum(edges)
    return jnp.concatenate([tc_out, sc_out.reshape(SC_SEGS, D)], axis=0)


# speedup vs baseline: 1.0262x; 1.0262x over previous
import functools

import jax
import jax.numpy as jnp
import numpy as np
from jax import lax
from jax.experimental import pallas as pl
from jax.experimental.pallas import tpu as pltpu
from jax.experimental.pallas import tpu_sc as plsc

NUM_SEGS = 512
ROWS_PER_SEG = 625
D = 128
NLANES = 16
NVEC = D // NLANES

NUM_CORES = 2
NUM_SUBCORES = 16
NUM_WORKERS = NUM_CORES * NUM_SUBCORES

TC_SEGS = 272
SC_SEGS = NUM_SEGS - TC_SEGS
SEGS_PER_WORKER = SC_SEGS // NUM_WORKERS
SC_BIG_WORKERS = SC_SEGS - SEGS_PER_WORKER * NUM_WORKERS
SEGS_PER_WORKER_MAX = SEGS_PER_WORKER + (1 if SC_BIG_WORKERS else 0)

NCHUNK = 5
CHUNK_ROWS = ROWS_PER_SEG // NCHUNK
CHUNK_ELEMS = CHUNK_ROWS * D

TC_BLOCK_SEGS = 8


def _sc_segment_sum(edges_flat):
    mesh = plsc.VectorSubcoreMesh(core_axis_name="c", subcore_axis_name="s")

    @functools.partial(
        pl.kernel,
        out_type=jax.ShapeDtypeStruct((SC_SEGS * D,), jnp.float32),
        mesh=mesh,
        scratch_types=(
            [pltpu.VMEM((CHUNK_ELEMS,), jnp.float32) for _ in range(NCHUNK)]
            + [pltpu.VMEM((SEGS_PER_WORKER_MAX * D,), jnp.float32)]
            + [pltpu.SemaphoreType.DMA for _ in range(NCHUNK)]
        ),
    )
    def body(edges_hbm, out_hbm, *scratch):
        bufs = scratch[:NCHUNK]
        stage = scratch[NCHUNK]
        sems = scratch[NCHUNK + 1:]
        cid = lax.axis_index("c")
        sid = lax.axis_index("s")
        wid = sid * NUM_CORES + cid
        is_big = wid < SC_BIG_WORKERS
        nseg = jnp.where(is_big, SEGS_PER_WORKER + 1, SEGS_PER_WORKER)
        base_seg = TC_SEGS + wid * SEGS_PER_WORKER + jnp.minimum(
            wid, SC_BIG_WORKERS
        )

        def chunk_copy(seg, b):
            e0 = (seg * ROWS_PER_SEG + b * CHUNK_ROWS) * D
            return pltpu.make_async_copy(
                edges_hbm.at[pl.ds(e0, CHUNK_ELEMS)], bufs[b], sems[b]
            )

        for b in range(NCHUNK):
            chunk_copy(base_seg, b).start()

        def seg_body(s, carry):
            seg = base_seg + s
            acc = tuple(jnp.zeros((NLANES,), jnp.float32) for _ in range(NVEC))
            for b in range(NCHUNK):
                chunk_copy(seg, b).wait()

                def row_body(r, a, b=b):
                    base = r * D
                    return tuple(
                        a[j] + bufs[b][pl.ds(base + j * NLANES, NLANES)]
                        for j in range(NVEC)
                    )

                acc = lax.fori_loop(0, CHUNK_ROWS, row_body, acc)

                @pl.when(s < nseg - 1)
                def _(b=b):
                    chunk_copy(seg + 1, b).start()

            for j in range(NVEC):
                stage[pl.ds(s * D + j * NLANES, NLANES)] = acc[j]
            return carry

        lax.fori_loop(0, nseg, seg_body, 0)
        local_out0 = (base_seg - TC_SEGS) * D

        @pl.when(is_big)
        def _():
            pltpu.sync_copy(
                stage.at[pl.ds(0, SEGS_PER_WORKER_MAX * D)],
                out_hbm.at[pl.ds(local_out0, SEGS_PER_WORKER_MAX * D)],
            )

        @pl.when(jnp.logical_not(is_big))
        def _():
            pltpu.sync_copy(
                stage.at[pl.ds(0, SEGS_PER_WORKER * D)],
                out_hbm.at[pl.ds(local_out0, SEGS_PER_WORKER * D)],
            )

    return body(edges_flat)


def _tc_body(x_ref, o_ref):
    o_ref[...] = jnp.stack(
        [
            jnp.sum(x_ref[g * ROWS_PER_SEG:(g + 1) * ROWS_PER_SEG, :], axis=0)
            for g in range(TC_BLOCK_SEGS)
        ]
    )


def _tc_segment_sum(edges2d):
    return pl.pallas_call(
        _tc_body,
        grid=(TC_SEGS // TC_BLOCK_SEGS,),
        in_specs=[
            pl.BlockSpec((TC_BLOCK_SEGS * ROWS_PER_SEG, D), lambda i: (i, 0))
        ],
        out_specs=pl.BlockSpec((TC_BLOCK_SEGS, D), lambda i: (i, 0)),
        out_shape=jax.ShapeDtypeStruct((TC_SEGS, D), jnp.float32),
    )(edges2d)


def kernel(edges, n_edge, num_hypergraphs):
    del n_edge, num_hypergraphs
    sc_out = _sc_segment_sum(edges.reshape(-1))
    tc_out = _tc_segment_sum(edges)
    return jnp.concatenate([tc_out, sc_out.reshape(SC_SEGS, D)], axis=0)
